# baseline (speedup 1.0000x reference)
import functools

import jax
import jax.numpy as jnp
import numpy as np
from jax import lax
from jax.experimental import pallas as pl
from jax.experimental.pallas import tpu as pltpu
from jax.experimental.pallas import tpu_sc as plsc

_B = 1024
_L = 200
_D = 128
_SCALE = float(np.sqrt(np.float32(_D)))

_NC = 2
_NS = 16
_NW = _NC * _NS
_SEQ_PER_W = _B // _NW
_NBUF = 4
_SPLIT = 152


def _positional_table():
    pos = np.arange(_L)[:, None].astype(np.float32)
    i = np.arange(_D)[None, :].astype(np.float32)
    angle_rates = 1.0 / np.power(
        10000.0, (2.0 * np.floor(i / 2.0)) / np.float32(_D))
    angles = pos * angle_rates
    pe = np.zeros((_L, _D), dtype=np.float32)
    pe[:, 0::2] = np.sin(angles[:, 0::2])
    pe[:, 1::2] = np.cos(angles[:, 1::2])
    return pe


_PE = _positional_table()


def _enc_kernel(idx_hbm, table_hbm, pe_hbm, out_hbm,
                pe_v, i0, i1, i2, i3, r0, r1, r2, r3, gsem, wsem, isem):
    wid = lax.axis_index("s") * _NC + lax.axis_index("c")
    rows = (r0, r1, r2, r3)
    idxb = (i0, i1, i2, i3)
    pltpu.sync_copy(pe_hbm, pe_v)
    seq0 = wid * _SEQ_PER_W

    def issue_idx(j, b):
        pltpu.async_copy(idx_hbm.at[pl.ds((seq0 + j) * _L, _L)], idxb[b],
                         isem.at[b])

    def wait_idx(b):
        pltpu.make_async_copy(idx_hbm.at[pl.ds(0, _L)], idxb[b],
                              isem.at[b]).wait()

    def issue_gather(b):
        pltpu.async_copy(table_hbm.at[idxb[b].at[pl.ds(0, 128)]],
                         rows[b].at[pl.ds(0, 128)], gsem.at[b])
        pltpu.async_copy(table_hbm.at[idxb[b].at[pl.ds(128, _L - 128)]],
                         rows[b].at[pl.ds(128, _L - 128)], gsem.at[b])

    def wait_gather(b):
        pltpu.make_async_copy(out_hbm.at[pl.ds(0, _L)], rows[b],
                              gsem.at[b]).wait()

    def issue_write_part(j, b, lo, n):
        pltpu.async_copy(rows[b].at[pl.ds(lo, n)],
                         out_hbm.at[pl.ds((seq0 + j) * _L + lo, n)],
                         wsem.at[b])

    def wait_write(b):
        pltpu.make_async_copy(rows[b].at[pl.ds(0, _SPLIT)],
                              out_hbm.at[pl.ds(0, _SPLIT)],
                              wsem.at[b]).wait()
        pltpu.make_async_copy(rows[b].at[pl.ds(0, _L - _SPLIT)],
                              out_hbm.at[pl.ds(0, _L - _SPLIT)],
                              wsem.at[b]).wait()

    def fuse_phase(ba, bb, lo, hi):
        ga = rows[ba]
        gb = rows[bb]

        def body(l, carry):
            for d in range(_D // 16):
                sl = pl.ds(d * 16, 16)
                pe_reg = pe_v[l, sl]
                ga[l, sl] = ga[l, sl] * _SCALE + pe_reg
                gb[l, sl] = gb[l, sl] * _SCALE + pe_reg
            return carry

        lax.fori_loop(lo, hi, body, 0)

    def step(j, b, wait_w, has_gather, has_idx, do_fuse):
        nb = (b + 2) % _NBUF
        if wait_w:
            wait_write(nb)
        if has_gather:
            wait_idx(nb)
            issue_gather(nb)
        wait_gather(b)
        if has_idx:
            issue_idx(j + 4, b)
        if do_fuse:
            pb = (b + _NBUF - 1) % _NBUF
            fuse_phase(pb, b, 0, _SPLIT)
            issue_write_part(j - 1, pb, 0, _SPLIT)
            issue_write_part(j, b, 0, _SPLIT)
            fuse_phase(pb, b, _SPLIT, _L)
            issue_write_part(j - 1, pb, _SPLIT, _L - _SPLIT)
            issue_write_part(j, b, _SPLIT, _L - _SPLIT)

    for b in range(_NBUF):
        issue_idx(b, b)
    wait_idx(0)
    wait_idx(1)
    issue_gather(0)
    issue_gather(1)

    for j in range(_NBUF):
        step(j, j, j >= 2, True, True, j % 2 == 1)

    def group(g, carry):
        j0 = g * _NBUF
        for u in range(_NBUF):
            step(j0 + u, u, True, True, True, u % 2 == 1)
        return carry

    lax.fori_loop(1, _SEQ_PER_W // _NBUF - 1, group, 0)
    step(28, 0, True, True, False, False)
    step(29, 1, True, True, False, True)
    step(30, 2, True, False, False, False)
    step(31, 3, True, False, False, True)
    wait_write(2)
    wait_write(3)


@jax.jit
def _encode(idx, emb_table, pe):
    mesh = plsc.VectorSubcoreMesh(core_axis_name="c", subcore_axis_name="s")
    f = functools.partial(
        pl.kernel,
        out_type=jax.ShapeDtypeStruct((_B * _L, _D), jnp.float32),
        mesh=mesh,
        scratch_types=[
            pltpu.VMEM((_L, _D), jnp.float32),
            pltpu.VMEM((_L,), jnp.int32),
            pltpu.VMEM((_L,), jnp.int32),
            pltpu.VMEM((_L,), jnp.int32),
            pltpu.VMEM((_L,), jnp.int32),
            pltpu.VMEM((_L, _D), jnp.float32),
            pltpu.VMEM((_L, _D), jnp.float32),
            pltpu.VMEM((_L, _D), jnp.float32),
            pltpu.VMEM((_L, _D), jnp.float32),
            pltpu.SemaphoreType.DMA((_NBUF,)),
            pltpu.SemaphoreType.DMA((_NBUF,)),
            pltpu.SemaphoreType.DMA((_NBUF,)),
        ],
    )(_enc_kernel)
    return f(idx, emb_table, pe)


def kernel(text, emb_table):
    idx = text.reshape(-1).astype(jnp.int32)
    out = _encode(idx, emb_table, _PE)
    return out.reshape(_B, _L, _D)
